# baseline (speedup 1.0000x reference)
import functools

import jax
import jax.numpy as jnp
from jax import lax
from jax.experimental import pallas as pl
from jax.experimental.pallas import tpu as pltpu
from jax.experimental.pallas import tpu_sc as plsc

_N = 10000
_E = 160000
_C = 128
_M0 = 3
_MALL = 9
_NSPEC = 90
_H = 64
_EIN = 128
_RESCALE = 5.0
_ROW = _MALL * _C

_NC = 2
_NS = 16

_NW = _NC * _NS
_EPW = 5008
_E_PAD = _EPW * _NW


def _species_body(ns_hbm, snd_hbm, rcv_hbm, outs_hbm, outr_hbm,
                  ns_v, snd_v, rcv_v, outs_v, outr_v):
    c = lax.axis_index("c")
    s = lax.axis_index("s")
    wid = s * _NC + c
    base = wid * _EPW
    pltpu.sync_copy(ns_hbm, ns_v)
    pltpu.sync_copy(snd_hbm.at[pl.ds(base, _EPW)], snd_v)
    pltpu.sync_copy(rcv_hbm.at[pl.ds(base, _EPW)], rcv_v)

    def body(i, carry):
        si = snd_v[pl.ds(i * 16, 16)]
        ri = rcv_v[pl.ds(i * 16, 16)]
        outs_v[pl.ds(i * 16, 16)] = plsc.load_gather(ns_v, [si])
        outr_v[pl.ds(i * 16, 16)] = plsc.load_gather(ns_v, [ri])
        return carry

    lax.fori_loop(0, _EPW // 16, body, 0)
    pltpu.sync_copy(outs_v, outs_hbm.at[pl.ds(base, _EPW)])
    pltpu.sync_copy(outr_v, outr_hbm.at[pl.ds(base, _EPW)])


_species_call = pl.kernel(
    _species_body,
    out_type=(jax.ShapeDtypeStruct((_E_PAD,), jnp.int32),
              jax.ShapeDtypeStruct((_E_PAD,), jnp.int32)),
    mesh=plsc.VectorSubcoreMesh(core_axis_name="c", subcore_axis_name="s"),
    compiler_params=pltpu.CompilerParams(needs_layout_passes=False),
    scratch_types=(
        pltpu.VMEM((_N,), jnp.int32),
        pltpu.VMEM((_EPW,), jnp.int32),
        pltpu.VMEM((_EPW,), jnp.int32),
        pltpu.VMEM((_EPW,), jnp.int32),
        pltpu.VMEM((_EPW,), jnp.int32),
    ),
)

_EB = 1280
_GRID = _E // _EB


def _ln_silu(x, g, b):
    mu = jnp.mean(x, axis=-1, keepdims=True)
    xc = x - mu
    var = jnp.mean(xc * xc, axis=-1, keepdims=True)
    y = xc * lax.rsqrt(var + 1e-5) * g + b
    return y * jax.nn.sigmoid(y)


def _mlp_body(ed_ref, sps_ref, spr_ref, w_ref, env_ref,
              W0d_ref, Ts2_ref, Tr2_ref, b0_ref, g0_ref, be0_ref,
              W1_ref, b1_ref, g1_ref, be1_ref, W2_ref, b2_ref, out_ref):
    f32 = jnp.float32
    iota = lax.broadcasted_iota(jnp.int32, (1, _NSPEC), 1)
    ohs = (sps_ref[...] == iota).astype(f32)
    ohr = (spr_ref[...] == iota).astype(f32)
    x = jnp.dot(ed_ref[...], W0d_ref[...], preferred_element_type=f32)
    x = x + jnp.dot(ohs, Ts2_ref[...], preferred_element_type=f32)
    x = x + jnp.dot(ohr, Tr2_ref[...], preferred_element_type=f32)
    x = x + b0_ref[...]
    x = _ln_silu(x, g0_ref[...], be0_ref[...])
    x = jnp.dot(x, W1_ref[...], preferred_element_type=f32) + b1_ref[...]
    x = _ln_silu(x, g1_ref[...], be1_ref[...])
    x = jnp.dot(x, W2_ref[...], preferred_element_type=f32) + b2_ref[...]
    x = x * (env_ref[...] * (1.0 / _RESCALE))
    w = w_ref[...]
    for i in range(_MALL):
        acc = (w[:, 9 * i + 0:9 * i + 1] * x[:, 0:_C]
               + w[:, 9 * i + 1:9 * i + 2] * x[:, _C:2 * _C]
               + w[:, 9 * i + 2:9 * i + 3] * x[:, 2 * _C:3 * _C])
        out_ref[:, i * _C:(i + 1) * _C] = acc


def _mlp_call(off, nblk, ed, sps, spr, w81, env, W0d, Ts2, Tr2, b0, g0, be0,
              W1, b1, g1, be1, W2, b2):
    full = lambda shape: pl.BlockSpec(shape, lambda i: (0, 0))
    blk = lambda shape: pl.BlockSpec(shape, lambda i: (i + off, 0))
    return pl.pallas_call(
        _mlp_body,
        grid=(nblk,),
        in_specs=[
            blk((_EB, _EIN)),
            blk((_EB, 1)),
            blk((_EB, 1)),
            blk((_EB, _MALL * _MALL)),
            blk((_EB, 1)),
            full((_EIN, _H)),
            full((_NSPEC, _H)),
            full((_NSPEC, _H)),
            full((1, _H)),
            full((1, _H)),
            full((1, _H)),
            full((_H, _H)),
            full((1, _H)),
            full((1, _H)),
            full((1, _H)),
            full((_H, _M0 * _C)),
            full((1, _M0 * _C)),
        ],
        out_specs=pl.BlockSpec((_EB, _ROW), lambda i: (i, 0)),
        out_shape=jax.ShapeDtypeStruct((nblk * _EB, _ROW), jnp.float32),
    )(ed, sps, spr, w81, env, W0d, Ts2, Tr2, b0, g0, be0,
      W1, b1, g1, be1, W2, b2)


_R = 1024
_N_OUT = _R * 10
_PASSES = 5
_ACC_NODES = 1040
_G = 16
_RCH = 2048
_SENT = _R
_OSTR = _R * _MALL // _NS


def _make_scatter(ept, e0, first):
    cap = ept + 32
    nfull, rem = divmod(ept, _RCH)
    sizes = [_RCH] * nfull + ([rem] if rem else [])

    def body(*refs):
        if first:
            ef_hbm, rcv_hbm, out_hbm = refs[:3]
            (acc_sh, pk_v, rch_v, eba_v, ebb_v, ixa_v, ixb_v,
             rfa_v, rfb_v, sema, semb) = refs[3:]
        else:
            ef_hbm, rcv_hbm, acc_in, out_hbm = refs[:4]
            (acc_sh, pk_v, rch_v, eba_v, ebb_v, ixa_v, ixb_v,
             rfa_v, rfb_v, sema, semb) = refs[4:]

        c = lax.axis_index("c")
        s = lax.axis_index("s")
        ebase = s * ept

        zero16 = jnp.zeros((16,), jnp.float32)
        lane = lax.iota(jnp.int32, 16)

        def fire(b, eb_ref, rf_ref, sem):
            pk16 = pk_v[pl.ds(b * _G, 16)]
            eb_ref[...] = lax.shift_right_logical(pk16, 11)
            return pltpu.async_copy(ef_hbm.at[eb_ref],
                                    rf_ref.reshape(_G, _ROW), sem)

        def scatter(b, rf_ref):
            pk16 = pk_v[pl.ds(b * _G, 16)]
            l9 = (pk16 & 2047) * _MALL
            for t in range(_MALL):
                r2 = 16 * t + lane
                idx = jnp.take(l9, r2 // _MALL) + (r2 % _MALL)
                if t < 8:
                    ixa_v[0, pl.ds(t * 16, 16)] = idx
                else:
                    ixb_v[0, :] = idx
            pltpu.sync_copy(rf_ref.at[pl.ds(0, 128)],
                            acc_sh.at[ixa_v.at[0]], add=True)
            pltpu.sync_copy(rf_ref.at[pl.ds(128, 16)],
                            acc_sh.at[ixb_v.at[0]], add=True)

        for r in range(_PASSES):
            base = (c * _PASSES + r) * _R

            if first:
                def zfill(i, carry):
                    rfa_v[i // 8, pl.ds((i % 8) * 16, 16)] = zero16
                    return carry

                lax.fori_loop(0, _MALL * _G * 8, zfill, 0)
                for k in range(4):
                    pltpu.sync_copy(rfa_v,
                                    acc_sh.at[pl.ds((s + 16 * k) * 144, 144)])

                @pl.when(s == 0)
                def _ztail():
                    pltpu.sync_copy(rfa_v, acc_sh.at[pl.ds(64 * 144, 144)])
            else:
                pltpu.sync_copy(
                    acc_in.at[pl.ds(base * _MALL + s * _OSTR, _OSTR)],
                    acc_sh.at[pl.ds(s * _OSTR, _OSTR)])

            plsc.subcore_barrier()

            def pre(i, carry):
                pk_v[pl.ds(i * 16, 16)] = jnp.full((16,), _SENT, jnp.int32)
                return carry

            lax.fori_loop(0, cap // 16, pre, 0)

            cnt = 0
            for ch, size in enumerate(sizes):
                pltpu.sync_copy(
                    rcv_hbm.at[pl.ds(e0 + ebase + ch * _RCH, size)],
                    rch_v.at[pl.ds(0, size)])
                off = ebase + ch * _RCH

                def scan(i, cnt, off=off):
                    rv = rch_v[pl.ds(i * 16, 16)]
                    m = (rv >= base) & (rv < base + _R)
                    pk = ((off + i * 16 + lane) << 11) | (rv - base)
                    plsc.store_compressed(pk_v.at[pl.ds(cnt, 16)], pk,
                                          mask=m)
                    return cnt + jnp.sum(m.astype(jnp.int32))

                cnt = lax.fori_loop(0, size // 16, scan, cnt)
            nb = (cnt + _G - 1) // _G

            fire(0, eba_v, rfa_v, sema)

            def pair(i, carry):
                ba, bb = 2 * i, 2 * i + 1
                fire(bb, ebb_v, rfb_v, semb)
                pltpu.make_async_copy(ef_hbm.at[eba_v],
                                      rfa_v.reshape(_G, _ROW), sema).wait()
                scatter(ba, rfa_v)
                fire(ba + 2, eba_v, rfa_v, sema)
                pltpu.make_async_copy(ef_hbm.at[ebb_v],
                                      rfb_v.reshape(_G, _ROW), semb).wait()

                @pl.when(bb < nb)
                def _odd():
                    scatter(bb, rfb_v)

                return carry

            lax.fori_loop(0, (nb + 1) // 2, pair, 0)
            pltpu.make_async_copy(ef_hbm.at[eba_v],
                                  rfa_v.reshape(_G, _ROW), sema).wait()
            plsc.subcore_barrier()

            pltpu.sync_copy(acc_sh.at[pl.ds(s * _OSTR, _OSTR)],
                            out_hbm.at[pl.ds(base * _MALL + s * _OSTR,
                                             _OSTR)])
            plsc.subcore_barrier()

    return pl.kernel(
        body,
        out_type=jax.ShapeDtypeStruct((_N_OUT * _MALL, _C), jnp.float32),
        mesh=plsc.VectorSubcoreMesh(core_axis_name="c", subcore_axis_name="s"),
        compiler_params=pltpu.CompilerParams(needs_layout_passes=False),
        scratch_types=(
            pltpu.VMEM_SHARED((_ACC_NODES * _MALL, _C), jnp.float32),
            pltpu.VMEM((cap,), jnp.int32),
            pltpu.VMEM((_RCH,), jnp.int32),
            pltpu.VMEM((_G,), jnp.int32),
            pltpu.VMEM((_G,), jnp.int32),
            pltpu.VMEM((1, 128), jnp.int32),
            pltpu.VMEM((1, 16), jnp.int32),
            pltpu.VMEM((_MALL * _G, _C), jnp.float32),
            pltpu.VMEM((_MALL * _G, _C), jnp.float32),
            pltpu.SemaphoreType.DMA,
            pltpu.SemaphoreType.DMA,
        ),
    )


_CHUNK_BLOCKS = (32, 32, 32, 29)


def kernel(node_species, edge_distances, senders, receivers, wigner_inv,
           edge_envelope, sender_table, recv_table, W0, b0, g0, be0,
           W1, b1, g1, be1, W2, b2):
    f32, i32 = jnp.float32, jnp.int32
    ns = node_species.astype(i32)
    snd = senders.astype(i32)
    rcv = receivers.astype(i32)
    pad = _E_PAD - _E
    snd_p = jnp.concatenate([snd, jnp.zeros((pad,), i32)])
    rcv_p = jnp.concatenate([rcv, jnp.zeros((pad,), i32)])
    sps_p, spr_p = _species_call(ns, snd_p, rcv_p)
    sps = sps_p[:_E].reshape(_E, 1)
    spr = spr_p[:_E].reshape(_E, 1)

    W0 = W0.astype(f32)
    Ts2 = sender_table.astype(f32) @ W0[_EIN:_EIN + _H]
    Tr2 = recv_table.astype(f32) @ W0[_EIN + _H:]
    W0d = W0[:_EIN]

    ed = edge_distances.astype(f32)
    w81 = wigner_inv.astype(f32).reshape(_E, _MALL * _MALL)
    env = edge_envelope.astype(f32).reshape(_E, 1)
    r2 = lambda v: v.astype(f32).reshape(1, -1)

    acc = None
    off = 0
    for nblk in _CHUNK_BLOCKS:
        ef_i = _mlp_call(off, nblk, ed, sps, spr, w81, env, W0d, Ts2, Tr2,
                         r2(b0), r2(g0), r2(be0),
                         W1.astype(f32), r2(b1), r2(g1), r2(be1),
                         W2.astype(f32), r2(b2))
        e0 = off * _EB
        scat = _make_scatter(nblk * _EB // _NS, e0, acc is None)
        acc = scat(ef_i, rcv) if acc is None else scat(ef_i, rcv, acc)
        off += nblk

    return acc.reshape(_N_OUT, _MALL, _C)[:_N]


# speedup vs baseline: 1.2068x; 1.2068x over previous
import functools

import jax
import jax.numpy as jnp
from jax import lax
from jax.experimental import pallas as pl
from jax.experimental.pallas import tpu as pltpu
from jax.experimental.pallas import tpu_sc as plsc

_N = 10000
_E = 160000
_C = 128
_M0 = 3
_MALL = 9
_NSPEC = 90
_H = 64
_EIN = 128
_RESCALE = 5.0
_ROW = _MALL * _C

_NC = 2
_NS = 16

_NW = _NC * _NS
_EPW = 5008
_E_PAD = _EPW * _NW


def _species_body(ns_hbm, snd_hbm, rcv_hbm, outs_hbm, outr_hbm,
                  ns_v, snd_v, rcv_v, outs_v, outr_v):
    c = lax.axis_index("c")
    s = lax.axis_index("s")
    wid = s * _NC + c
    base = wid * _EPW
    pltpu.sync_copy(ns_hbm, ns_v)
    pltpu.sync_copy(snd_hbm.at[pl.ds(base, _EPW)], snd_v)
    pltpu.sync_copy(rcv_hbm.at[pl.ds(base, _EPW)], rcv_v)

    def body(i, carry):
        si = snd_v[pl.ds(i * 16, 16)]
        ri = rcv_v[pl.ds(i * 16, 16)]
        outs_v[pl.ds(i * 16, 16)] = plsc.load_gather(ns_v, [si])
        outr_v[pl.ds(i * 16, 16)] = plsc.load_gather(ns_v, [ri])
        return carry

    lax.fori_loop(0, _EPW // 16, body, 0)
    pltpu.sync_copy(outs_v, outs_hbm.at[pl.ds(base, _EPW)])
    pltpu.sync_copy(outr_v, outr_hbm.at[pl.ds(base, _EPW)])


_species_call = pl.kernel(
    _species_body,
    out_type=(jax.ShapeDtypeStruct((_E_PAD,), jnp.int32),
              jax.ShapeDtypeStruct((_E_PAD,), jnp.int32)),
    mesh=plsc.VectorSubcoreMesh(core_axis_name="c", subcore_axis_name="s"),
    compiler_params=pltpu.CompilerParams(needs_layout_passes=False),
    scratch_types=(
        pltpu.VMEM((_N,), jnp.int32),
        pltpu.VMEM((_EPW,), jnp.int32),
        pltpu.VMEM((_EPW,), jnp.int32),
        pltpu.VMEM((_EPW,), jnp.int32),
        pltpu.VMEM((_EPW,), jnp.int32),
    ),
)

_EB = 1280
_GRID = _E // _EB


def _ln_silu(x, g, b):
    mu = jnp.mean(x, axis=-1, keepdims=True)
    xc = x - mu
    var = jnp.mean(xc * xc, axis=-1, keepdims=True)
    y = xc * lax.rsqrt(var + 1e-5) * g + b
    return y * jax.nn.sigmoid(y)


def _mlp_body(ed_ref, sps_ref, spr_ref, w_ref, env_ref,
              W0d_ref, Ts2_ref, Tr2_ref, b0_ref, g0_ref, be0_ref,
              W1_ref, b1_ref, g1_ref, be1_ref, W2_ref, b2_ref, out_ref):
    f32 = jnp.float32
    iota = lax.broadcasted_iota(jnp.int32, (1, _NSPEC), 1)
    ohs = (sps_ref[...] == iota).astype(f32)
    ohr = (spr_ref[...] == iota).astype(f32)
    x = jnp.dot(ed_ref[...], W0d_ref[...], preferred_element_type=f32)
    x = x + jnp.dot(ohs, Ts2_ref[...], preferred_element_type=f32)
    x = x + jnp.dot(ohr, Tr2_ref[...], preferred_element_type=f32)
    x = x + b0_ref[...]
    x = _ln_silu(x, g0_ref[...], be0_ref[...])
    x = jnp.dot(x, W1_ref[...], preferred_element_type=f32) + b1_ref[...]
    x = _ln_silu(x, g1_ref[...], be1_ref[...])
    x = jnp.dot(x, W2_ref[...], preferred_element_type=f32) + b2_ref[...]
    x = x * (env_ref[...] * (1.0 / _RESCALE))
    w = w_ref[...]
    for i in range(_MALL):
        acc = (w[:, 9 * i + 0:9 * i + 1] * x[:, 0:_C]
               + w[:, 9 * i + 1:9 * i + 2] * x[:, _C:2 * _C]
               + w[:, 9 * i + 2:9 * i + 3] * x[:, 2 * _C:3 * _C])
        out_ref[:, i * _C:(i + 1) * _C] = acc


def _mlp_call(off, nblk, ed, sps, spr, w81, env, W0d, Ts2, Tr2, b0, g0, be0,
              W1, b1, g1, be1, W2, b2):
    full = lambda shape: pl.BlockSpec(shape, lambda i: (0, 0))
    blk = lambda shape: pl.BlockSpec(shape, lambda i: (i + off, 0))
    return pl.pallas_call(
        _mlp_body,
        grid=(nblk,),
        in_specs=[
            blk((_EB, _EIN)),
            blk((_EB, 1)),
            blk((_EB, 1)),
            blk((_EB, _MALL * _MALL)),
            blk((_EB, 1)),
            full((_EIN, _H)),
            full((_NSPEC, _H)),
            full((_NSPEC, _H)),
            full((1, _H)),
            full((1, _H)),
            full((1, _H)),
            full((_H, _H)),
            full((1, _H)),
            full((1, _H)),
            full((1, _H)),
            full((_H, _M0 * _C)),
            full((1, _M0 * _C)),
        ],
        out_specs=pl.BlockSpec((_EB, _ROW), lambda i: (i, 0)),
        out_shape=jax.ShapeDtypeStruct((nblk * _EB, _ROW), jnp.float32),
    )(ed, sps, spr, w81, env, W0d, Ts2, Tr2, b0, g0, be0,
      W1, b1, g1, be1, W2, b2)


_R = 1280
_N_OUT = _R * 8
_PASSES = 4
_ACC_NODES = 1296
_G = 16
_RCH = 2048
_SENT = _R
_OSTR = _R * _MALL // _NS


def _make_scatter(ept, e0, first):
    cap = ept + 16
    nfull, rem = divmod(ept, _RCH)
    sizes = [_RCH] * nfull + ([rem] if rem else [])

    def body(*refs):
        if first:
            ef_hbm, rcv_hbm, out_hbm = refs[:3]
            acc_sh, pk_v, rch_v, eb_v, ixa_v, ixb_v, rf_v, sem = refs[3:]
        else:
            ef_hbm, rcv_hbm, acc_in, out_hbm = refs[:4]
            acc_sh, pk_v, rch_v, eb_v, ixa_v, ixb_v, rf_v, sem = refs[4:]

        c = lax.axis_index("c")
        s = lax.axis_index("s")
        ebase = s * ept

        zero16 = jnp.zeros((16,), jnp.float32)
        lane = lax.iota(jnp.int32, 16)

        for r in range(_PASSES):
            base = (c * _PASSES + r) * _R

            if first:
                def zfill(i, carry):
                    rf_v[i // 8, pl.ds((i % 8) * 16, 16)] = zero16
                    return carry

                lax.fori_loop(0, _MALL * _G * 8, zfill, 0)
                for k in range(5):
                    pltpu.sync_copy(rf_v,
                                    acc_sh.at[pl.ds((s + 16 * k) * 144, 144)])

                @pl.when(s == 0)
                def _ztail():
                    pltpu.sync_copy(rf_v, acc_sh.at[pl.ds(80 * 144, 144)])
            else:
                pltpu.sync_copy(
                    acc_in.at[pl.ds(base * _MALL + s * _OSTR, _OSTR)],
                    acc_sh.at[pl.ds(s * _OSTR, _OSTR)])

            plsc.subcore_barrier()

            def pre(i, carry):
                pk_v[pl.ds(i * 16, 16)] = jnp.full((16,), _SENT, jnp.int32)
                return carry

            lax.fori_loop(0, cap // 16, pre, 0)

            cnt = 0
            for ch, size in enumerate(sizes):
                pltpu.sync_copy(
                    rcv_hbm.at[pl.ds(e0 + ebase + ch * _RCH, size)],
                    rch_v.at[pl.ds(0, size)])
                off = ebase + ch * _RCH

                def scan(i, cnt, off=off):
                    rv = rch_v[pl.ds(i * 16, 16)]
                    m = (rv >= base) & (rv < base + _R)
                    pk = ((off + i * 16 + lane) << 11) | (rv - base)
                    plsc.store_compressed(pk_v.at[pl.ds(cnt, 16)], pk,
                                          mask=m)
                    return cnt + jnp.sum(m.astype(jnp.int32))

                cnt = lax.fori_loop(0, size // 16, scan, cnt)
            nb = (cnt + _G - 1) // _G

            def gs(b, carry):
                pk16 = pk_v[pl.ds(b * _G, 16)]
                eb_v[...] = lax.shift_right_logical(pk16, 11)
                pltpu.async_copy(ef_hbm.at[eb_v],
                                 rf_v.reshape(_G, _ROW), sem).wait()
                l9 = (pk16 & 2047) * _MALL
                for t in range(_MALL):
                    r2 = 16 * t + lane
                    idx = jnp.take(l9, r2 // _MALL) + (r2 % _MALL)
                    if t < 8:
                        ixa_v[0, pl.ds(t * 16, 16)] = idx
                    else:
                        ixb_v[0, :] = idx
                pltpu.sync_copy(rf_v.at[pl.ds(0, 128)],
                                acc_sh.at[ixa_v.at[0]], add=True)
                pltpu.sync_copy(rf_v.at[pl.ds(128, 16)],
                                acc_sh.at[ixb_v.at[0]], add=True)
                return carry

            lax.fori_loop(0, nb, gs, 0)
            plsc.subcore_barrier()

            pltpu.sync_copy(acc_sh.at[pl.ds(s * _OSTR, _OSTR)],
                            out_hbm.at[pl.ds(base * _MALL + s * _OSTR,
                                             _OSTR)])
            plsc.subcore_barrier()

    return pl.kernel(
        body,
        out_type=jax.ShapeDtypeStruct((_N_OUT * _MALL, _C), jnp.float32),
        mesh=plsc.VectorSubcoreMesh(core_axis_name="c", subcore_axis_name="s"),
        compiler_params=pltpu.CompilerParams(needs_layout_passes=False),
        scratch_types=(
            pltpu.VMEM_SHARED((_ACC_NODES * _MALL, _C), jnp.float32),
            pltpu.VMEM((cap,), jnp.int32),
            pltpu.VMEM((_RCH,), jnp.int32),
            pltpu.VMEM((_G,), jnp.int32),
            pltpu.VMEM((1, 128), jnp.int32),
            pltpu.VMEM((1, 16), jnp.int32),
            pltpu.VMEM((_MALL * _G, _C), jnp.float32),
            pltpu.SemaphoreType.DMA,
        ),
    )


_CHUNK_BLOCKS = (32, 32, 32, 29)


def kernel(node_species, edge_distances, senders, receivers, wigner_inv,
           edge_envelope, sender_table, recv_table, W0, b0, g0, be0,
           W1, b1, g1, be1, W2, b2):
    f32, i32 = jnp.float32, jnp.int32
    ns = node_species.astype(i32)
    snd = senders.astype(i32)
    rcv = receivers.astype(i32)
    pad = _E_PAD - _E
    snd_p = jnp.concatenate([snd, jnp.zeros((pad,), i32)])
    rcv_p = jnp.concatenate([rcv, jnp.zeros((pad,), i32)])
    sps_p, spr_p = _species_call(ns, snd_p, rcv_p)
    sps = sps_p[:_E].reshape(_E, 1)
    spr = spr_p[:_E].reshape(_E, 1)

    W0 = W0.astype(f32)
    Ts2 = sender_table.astype(f32) @ W0[_EIN:_EIN + _H]
    Tr2 = recv_table.astype(f32) @ W0[_EIN + _H:]
    W0d = W0[:_EIN]

    ed = edge_distances.astype(f32)
    w81 = wigner_inv.astype(f32).reshape(_E, _MALL * _MALL)
    env = edge_envelope.astype(f32).reshape(_E, 1)
    r2 = lambda v: v.astype(f32).reshape(1, -1)

    acc = None
    off = 0
    for nblk in _CHUNK_BLOCKS:
        ef_i = _mlp_call(off, nblk, ed, sps, spr, w81, env, W0d, Ts2, Tr2,
                         r2(b0), r2(g0), r2(be0),
                         W1.astype(f32), r2(b1), r2(g1), r2(be1),
                         W2.astype(f32), r2(b2))
        e0 = off * _EB
        scat = _make_scatter(nblk * _EB // _NS, e0, acc is None)
        acc = scat(ef_i, rcv) if acc is None else scat(ef_i, rcv, acc)
        off += nblk

    return acc.reshape(_N_OUT, _MALL, _C)[:_N]


# speedup vs baseline: 1.2359x; 1.0241x over previous
import functools

import jax
import jax.numpy as jnp
from jax import lax
from jax.experimental import pallas as pl
from jax.experimental.pallas import tpu as pltpu
from jax.experimental.pallas import tpu_sc as plsc

_N = 10000
_E = 160000
_C = 128
_M0 = 3
_MALL = 9
_NSPEC = 90
_H = 64
_EIN = 128
_RESCALE = 5.0
_ROW = _MALL * _C

_NC = 2
_NS = 16

_NW = _NC * _NS
_EPW = 5008
_E_PAD = _EPW * _NW


def _species_body(ns_hbm, snd_hbm, rcv_hbm, outs_hbm, outr_hbm,
                  ns_v, snd_v, rcv_v, outs_v, outr_v):
    c = lax.axis_index("c")
    s = lax.axis_index("s")
    wid = s * _NC + c
    base = wid * _EPW
    pltpu.sync_copy(ns_hbm, ns_v)
    pltpu.sync_copy(snd_hbm.at[pl.ds(base, _EPW)], snd_v)
    pltpu.sync_copy(rcv_hbm.at[pl.ds(base, _EPW)], rcv_v)

    def body(i, carry):
        si = snd_v[pl.ds(i * 16, 16)]
        ri = rcv_v[pl.ds(i * 16, 16)]
        outs_v[pl.ds(i * 16, 16)] = plsc.load_gather(ns_v, [si])
        outr_v[pl.ds(i * 16, 16)] = plsc.load_gather(ns_v, [ri])
        return carry

    lax.fori_loop(0, _EPW // 16, body, 0)
    pltpu.sync_copy(outs_v, outs_hbm.at[pl.ds(base, _EPW)])
    pltpu.sync_copy(outr_v, outr_hbm.at[pl.ds(base, _EPW)])


_species_call = pl.kernel(
    _species_body,
    out_type=(jax.ShapeDtypeStruct((_E_PAD,), jnp.int32),
              jax.ShapeDtypeStruct((_E_PAD,), jnp.int32)),
    mesh=plsc.VectorSubcoreMesh(core_axis_name="c", subcore_axis_name="s"),
    compiler_params=pltpu.CompilerParams(needs_layout_passes=False),
    scratch_types=(
        pltpu.VMEM((_N,), jnp.int32),
        pltpu.VMEM((_EPW,), jnp.int32),
        pltpu.VMEM((_EPW,), jnp.int32),
        pltpu.VMEM((_EPW,), jnp.int32),
        pltpu.VMEM((_EPW,), jnp.int32),
    ),
)

_EB = 1280
_GRID = _E // _EB


def _ln_silu(x, g, b):
    mu = jnp.mean(x, axis=-1, keepdims=True)
    xc = x - mu
    var = jnp.mean(xc * xc, axis=-1, keepdims=True)
    y = xc * lax.rsqrt(var + 1e-5) * g + b
    return y * jax.nn.sigmoid(y)


def _mlp_body(ed_ref, sps_ref, spr_ref, w_ref, env_ref,
              W0d_ref, Ts2_ref, Tr2_ref, b0_ref, g0_ref, be0_ref,
              W1_ref, b1_ref, g1_ref, be1_ref, W2_ref, b2_ref, out_ref):
    f32 = jnp.float32
    iota = lax.broadcasted_iota(jnp.int32, (1, _NSPEC), 1)
    ohs = (sps_ref[...] == iota).astype(f32)
    ohr = (spr_ref[...] == iota).astype(f32)
    x = jnp.dot(ed_ref[...], W0d_ref[...], preferred_element_type=f32)
    x = x + jnp.dot(ohs, Ts2_ref[...], preferred_element_type=f32)
    x = x + jnp.dot(ohr, Tr2_ref[...], preferred_element_type=f32)
    x = x + b0_ref[...]
    x = _ln_silu(x, g0_ref[...], be0_ref[...])
    x = jnp.dot(x, W1_ref[...], preferred_element_type=f32) + b1_ref[...]
    x = _ln_silu(x, g1_ref[...], be1_ref[...])
    x = jnp.dot(x, W2_ref[...], preferred_element_type=f32) + b2_ref[...]
    x = x * (env_ref[...] * (1.0 / _RESCALE))
    w = w_ref[...]
    for i in range(_MALL):
        acc = (w[:, 9 * i + 0:9 * i + 1] * x[:, 0:_C]
               + w[:, 9 * i + 1:9 * i + 2] * x[:, _C:2 * _C]
               + w[:, 9 * i + 2:9 * i + 3] * x[:, 2 * _C:3 * _C])
        out_ref[:, i * _C:(i + 1) * _C] = acc


def _mlp_call(off, nblk, ed, sps, spr, w81, env, W0d, Ts2, Tr2, b0, g0, be0,
              W1, b1, g1, be1, W2, b2):
    full = lambda shape: pl.BlockSpec(shape, lambda i: (0, 0))
    blk = lambda shape: pl.BlockSpec(shape, lambda i: (i + off, 0))
    return pl.pallas_call(
        _mlp_body,
        grid=(nblk,),
        in_specs=[
            blk((_EB, _EIN)),
            blk((_EB, 1)),
            blk((_EB, 1)),
            blk((_EB, _MALL * _MALL)),
            blk((_EB, 1)),
            full((_EIN, _H)),
            full((_NSPEC, _H)),
            full((_NSPEC, _H)),
            full((1, _H)),
            full((1, _H)),
            full((1, _H)),
            full((_H, _H)),
            full((1, _H)),
            full((1, _H)),
            full((1, _H)),
            full((_H, _M0 * _C)),
            full((1, _M0 * _C)),
        ],
        out_specs=pl.BlockSpec((_EB, _ROW), lambda i: (i, 0)),
        out_shape=jax.ShapeDtypeStruct((nblk * _EB, _ROW), jnp.float32),
    )(ed, sps, spr, w81, env, W0d, Ts2, Tr2, b0, g0, be0,
      W1, b1, g1, be1, W2, b2)


_R = 1280
_N_OUT = _R * 8
_PASSES = 4
_ACC_NODES = 1296
_G = 16
_RCH = 2048
_SENT = _R
_OSTR = _R * _MALL // _NS


def _make_scatter(ept, e0, first):
    cap = ept + 16
    nfull, rem = divmod(ept, _RCH)
    sizes = [_RCH] * nfull + ([rem] if rem else [])

    def body(*refs):
        if first:
            ef_hbm, rcv_hbm, out_hbm = refs[:3]
            acc_sh, pk_v, rch_v, eb_v, ixa_v, ixb_v, rf_v, sem = refs[3:]
        else:
            ef_hbm, rcv_hbm, acc_in, out_hbm = refs[:4]
            acc_sh, pk_v, rch_v, eb_v, ixa_v, ixb_v, rf_v, sem = refs[4:]

        c = lax.axis_index("c")
        s = lax.axis_index("s")
        ebase = s * ept

        zero16 = jnp.zeros((16,), jnp.float32)
        lane = lax.iota(jnp.int32, 16)

        for r in range(_PASSES):
            base = (c * _PASSES + r) * _R

            if first:
                def zfill(i, carry):
                    rf_v[i // 8, pl.ds((i % 8) * 16, 16)] = zero16
                    return carry

                lax.fori_loop(0, _MALL * _G * 8, zfill, 0)
                for k in range(5):
                    pltpu.sync_copy(rf_v,
                                    acc_sh.at[pl.ds((s + 16 * k) * 144, 144)])

                @pl.when(s == 0)
                def _ztail():
                    pltpu.sync_copy(rf_v, acc_sh.at[pl.ds(80 * 144, 144)])
            else:
                pltpu.sync_copy(
                    acc_in.at[pl.ds(base + s * 80, 80)],
                    acc_sh.reshape(_ACC_NODES, _MALL, _C).at[pl.ds(s * 80,
                                                                   80)])

            plsc.subcore_barrier()

            def pre(i, carry):
                pk_v[pl.ds(i * 16, 16)] = jnp.full((16,), _SENT, jnp.int32)
                return carry

            lax.fori_loop(0, cap // 16, pre, 0)

            cnt = 0
            for ch, size in enumerate(sizes):
                pltpu.sync_copy(
                    rcv_hbm.at[pl.ds(e0 + ebase + ch * _RCH, size)],
                    rch_v.at[pl.ds(0, size)])
                off = ebase + ch * _RCH

                def scan(i, cnt, off=off):
                    rv = rch_v[pl.ds(i * 16, 16)]
                    m = (rv >= base) & (rv < base + _R)
                    pk = ((off + i * 16 + lane) << 11) | (rv - base)
                    plsc.store_compressed(pk_v.at[pl.ds(cnt, 16)], pk,
                                          mask=m)
                    return cnt + jnp.sum(m.astype(jnp.int32))

                cnt = lax.fori_loop(0, size // 16, scan, cnt)
            nb = (cnt + _G - 1) // _G

            def gs(b, carry):
                pk16 = pk_v[pl.ds(b * _G, 16)]
                eb_v[...] = lax.shift_right_logical(pk16, 11)
                pltpu.async_copy(ef_hbm.at[eb_v],
                                 rf_v.reshape(_G, _ROW), sem).wait()
                l9 = (pk16 & 2047) * _MALL
                for t in range(_MALL):
                    r2 = 16 * t + lane
                    idx = jnp.take(l9, r2 // _MALL) + (r2 % _MALL)
                    if t < 8:
                        ixa_v[0, pl.ds(t * 16, 16)] = idx
                    else:
                        ixb_v[0, :] = idx
                pltpu.sync_copy(rf_v.at[pl.ds(0, 128)],
                                acc_sh.at[ixa_v.at[0]], add=True)
                pltpu.sync_copy(rf_v.at[pl.ds(128, 16)],
                                acc_sh.at[ixb_v.at[0]], add=True)
                return carry

            lax.fori_loop(0, nb, gs, 0)
            plsc.subcore_barrier()

            pltpu.sync_copy(
                acc_sh.reshape(_ACC_NODES, _MALL, _C).at[pl.ds(s * 80, 80)],
                out_hbm.at[pl.ds(base + s * 80, 80)])
            plsc.subcore_barrier()

    return pl.kernel(
        body,
        out_type=jax.ShapeDtypeStruct((_N_OUT, _MALL, _C), jnp.float32),
        mesh=plsc.VectorSubcoreMesh(core_axis_name="c", subcore_axis_name="s"),
        compiler_params=pltpu.CompilerParams(needs_layout_passes=False),
        scratch_types=(
            pltpu.VMEM_SHARED((_ACC_NODES * _MALL, _C), jnp.float32),
            pltpu.VMEM((cap,), jnp.int32),
            pltpu.VMEM((_RCH,), jnp.int32),
            pltpu.VMEM((_G,), jnp.int32),
            pltpu.VMEM((1, 128), jnp.int32),
            pltpu.VMEM((1, 16), jnp.int32),
            pltpu.VMEM((_MALL * _G, _C), jnp.float32),
            pltpu.SemaphoreType.DMA,
        ),
    )


_CHUNK_BLOCKS = (32, 32, 32, 29)


def kernel(node_species, edge_distances, senders, receivers, wigner_inv,
           edge_envelope, sender_table, recv_table, W0, b0, g0, be0,
           W1, b1, g1, be1, W2, b2):
    f32, i32 = jnp.float32, jnp.int32
    ns = node_species.astype(i32)
    snd = senders.astype(i32)
    rcv = receivers.astype(i32)
    pad = _E_PAD - _E
    snd_p = jnp.concatenate([snd, jnp.zeros((pad,), i32)])
    rcv_p = jnp.concatenate([rcv, jnp.zeros((pad,), i32)])
    sps_p, spr_p = _species_call(ns, snd_p, rcv_p)
    sps = sps_p[:_E].reshape(_E, 1)
    spr = spr_p[:_E].reshape(_E, 1)

    W0 = W0.astype(f32)
    Ts2 = sender_table.astype(f32) @ W0[_EIN:_EIN + _H]
    Tr2 = recv_table.astype(f32) @ W0[_EIN + _H:]
    W0d = W0[:_EIN]

    ed = edge_distances.astype(f32)
    w81 = wigner_inv.astype(f32).reshape(_E, _MALL * _MALL)
    env = edge_envelope.astype(f32).reshape(_E, 1)
    r2 = lambda v: v.astype(f32).reshape(1, -1)

    acc = None
    off = 0
    for nblk in _CHUNK_BLOCKS:
        ef_i = _mlp_call(off, nblk, ed, sps, spr, w81, env, W0d, Ts2, Tr2,
                         r2(b0), r2(g0), r2(be0),
                         W1.astype(f32), r2(b1), r2(g1), r2(be1),
                         W2.astype(f32), r2(b2))
        e0 = off * _EB
        scat = _make_scatter(nblk * _EB // _NS, e0, acc is None)
        acc = scat(ef_i, rcv) if acc is None else scat(ef_i, rcv, acc)
        off += nblk

    return acc[:_N]
